# baseline (speedup 1.0000x reference)
import functools

import jax
import jax.numpy as jnp
from jax import lax
from jax.experimental import pallas as pl
from jax.experimental.pallas import tpu as pltpu
from jax.experimental.pallas import tpu_sc as plsc

B, P, F = 4, 8192, 768
ROWS = B * P
NC, NS = 2, 16
NW = NC * NS
RPW = ROWS // NW
NBUF = 4
CHUNK = 32
NCHUNK = RPW // CHUNK

_mesh = plsc.VectorSubcoreMesh(core_axis_name="c", subcore_axis_name="s")


@functools.partial(
    pl.kernel,
    out_type=jax.ShapeDtypeStruct((ROWS, F), jnp.float32),
    mesh=_mesh,
    scratch_types=[
        pltpu.VMEM((RPW,), jnp.int32),
        pltpu.VMEM((NBUF, CHUNK, F), jnp.float32),
        [pltpu.SemaphoreType.DMA] * NBUF,
        [pltpu.SemaphoreType.DMA] * NBUF,
    ],
)
def _gather_kernel(x_hbm, idx_hbm, out_hbm, idx_v, rows_v, gsems, ssems):
    wid = lax.axis_index("s") * NC + lax.axis_index("c")
    base = wid * RPW

    def fire_gather(c, b):
        pltpu.async_copy(
            x_hbm.at[idx_v.at[pl.ds(c * CHUNK, CHUNK)]], rows_v.at[b], gsems[b]
        )

    def wait_gather(b):
        pltpu.make_async_copy(
            x_hbm.at[idx_v.at[pl.ds(0, CHUNK)]], rows_v.at[b], gsems[b]
        ).wait()

    def fire_store(c, b):
        pltpu.async_copy(
            rows_v.at[b], out_hbm.at[pl.ds(base + c * CHUNK, CHUNK)], ssems[b]
        )

    def wait_store(b):
        pltpu.make_async_copy(
            rows_v.at[b], out_hbm.at[pl.ds(base, CHUNK)], ssems[b]
        ).wait()

    pltpu.sync_copy(idx_hbm.at[pl.ds(base, RPW)], idx_v)

    fire_gather(0, 0)
    fire_gather(1, 1)

    @pl.loop(0, NCHUNK, step=NBUF)
    def _(k):
        for j in range(NBUF):
            c = k + j
            nb = (j + 2) % NBUF
            wait_gather(j)
            fire_store(c, j)

            @pl.when(c + 2 < NCHUNK)
            def _():
                @pl.when(c >= 2)
                def _():
                    wait_store(nb)

                fire_gather(c + 2, nb)

    for j in range(NBUF):
        wait_store(j)


def kernel(x, routing_map):
    x_flat = x.reshape(ROWS, F)
    idx_flat = (
        routing_map[None, :] + (P * jnp.arange(B, dtype=jnp.int32))[:, None]
    ).reshape(ROWS)
    out_flat = _gather_kernel(x_flat, idx_flat)
    return out_flat.reshape(B, P, F)


# speedup vs baseline: 1.0109x; 1.0109x over previous
import functools

import jax
import jax.numpy as jnp
from jax import lax
from jax.experimental import pallas as pl
from jax.experimental.pallas import tpu as pltpu
from jax.experimental.pallas import tpu_sc as plsc

B, P, F = 4, 8192, 768
NC, NS = 2, 16
NW = NC * NS
WPB = NW // B
RPW = P // WPB
CHUNK = 64
NCHUNK = RPW // CHUNK

_mesh = plsc.VectorSubcoreMesh(core_axis_name="c", subcore_axis_name="s")


@functools.partial(
    pl.kernel,
    out_type=jax.ShapeDtypeStruct((B, P, F), jnp.float32),
    mesh=_mesh,
    scratch_types=[
        pltpu.VMEM((RPW,), jnp.int32),
        pltpu.VMEM((2, CHUNK, F), jnp.float32),
        pltpu.SemaphoreType.DMA,
    ],
)
def _gather_kernel(x_hbm, idx_hbm, out_hbm, idx_v, rows_v, gsem):
    wid = lax.axis_index("s") * NC + lax.axis_index("c")
    bb = wid // WPB
    pbase = (wid % WPB) * RPW

    pltpu.sync_copy(idx_hbm.at[pl.ds(pbase, RPW)], idx_v)

    def fire_gather(c, b):
        pltpu.async_copy(
            x_hbm.at[bb].at[idx_v.at[pl.ds(c * CHUNK, CHUNK)]],
            rows_v.at[b],
            gsem,
        )

    fire_gather(0, 0)
    fire_gather(1, 1)

    @pl.loop(0, NCHUNK, step=2)
    def _(k):
        for b in range(2):
            c = k + b
            pltpu.make_async_copy(
                x_hbm.at[bb].at[idx_v.at[pl.ds(0, CHUNK)]], rows_v.at[b], gsem
            ).wait()
            pltpu.sync_copy(
                rows_v.at[b], out_hbm.at[bb].at[pl.ds(pbase + c * CHUNK, CHUNK)]
            )

            @pl.when(c + 2 < NCHUNK)
            def _():
                fire_gather(c + 2, b)


def kernel(x, routing_map):
    return _gather_kernel(x, routing_map)
